# baseline (speedup 1.0000x reference)
import functools
import math

import jax
import jax.numpy as jnp
from jax import lax
from jax.experimental import pallas as pl
from jax.experimental.pallas import tpu as pltpu

N_EXPERTS = 16
N_TOKENS = 8192
EMBED = 2048
CAP = 1024
TOK_BLK = 512
N_BLK = N_TOKENS // TOK_BLK


def _tc_gate_body(hs_ref, w_ref, scores_ref, tbits_ref):
    i = pl.program_id(0)
    logits = lax.dot_general(
        w_ref[...], hs_ref[...], (((1,), (1,)), ((), ())),
        preferred_element_type=jnp.float32)
    scores_ref[:, pl.ds(i * TOK_BLK, TOK_BLK)] = jax.nn.sigmoid(logits)

    @pl.when(i == N_BLK - 1)
    def _():
        bits = lax.bitcast_convert_type(scores_ref[...], jnp.int32)

        def step(_, lohi):
            lo, hi = lohi
            mid = (lo + hi) >> 1
            cnt = jnp.sum((bits >= mid).astype(jnp.int32), axis=1,
                          keepdims=True)
            ge = cnt >= CAP
            return jnp.where(ge, mid, lo), jnp.where(ge, hi, mid)

        lo0 = jnp.zeros((N_EXPERTS, 1), jnp.int32)
        hi0 = jnp.full((N_EXPERTS, 1), 0x3F800001, jnp.int32)
        lo, _ = lax.fori_loop(0, 30, step, (lo0, hi0))
        tbits_ref[...] = jnp.broadcast_to(lo, (N_EXPERTS, 128))


_tc_gate = pl.pallas_call(
    _tc_gate_body,
    grid=(N_BLK,),
    in_specs=[
        pl.BlockSpec((TOK_BLK, EMBED), lambda i: (i, 0)),
        pl.BlockSpec((N_EXPERTS, EMBED), lambda i: (0, 0)),
    ],
    out_specs=[
        pl.BlockSpec((N_EXPERTS, N_TOKENS), lambda i: (0, 0)),
        pl.BlockSpec((N_EXPERTS, 128), lambda i: (0, 0)),
    ],
    out_shape=[
        jax.ShapeDtypeStruct((N_EXPERTS, N_TOKENS), jnp.float32),
        jax.ShapeDtypeStruct((N_EXPERTS, 128), jnp.int32),
    ],
)


def kernel(hidden_states, weight):
    hs = hidden_states.reshape(-1, EMBED)
    scores, tbits = _tc_gate(hs, weight)
    del tbits
    topk_score, topk_ids = lax.top_k(scores, CAP)
    return topk_ids, topk_score


# speedup vs baseline: 1.3196x; 1.3196x over previous
import functools
import math

import jax
import jax.numpy as jnp
from jax import lax
from jax.experimental import pallas as pl
from jax.experimental.pallas import tpu as pltpu
from jax.experimental.pallas import tpu_sc as plsc

N_EXPERTS = 16
N_TOKENS = 8192
EMBED = 2048
CAP = 1024
TOK_BLK = 512
N_BLK = N_TOKENS // TOK_BLK
SELCAP = N_TOKENS + 16


def _tc_gate_body(hs_ref, w_ref, scores_ref, tbits_ref):
    i = pl.program_id(0)
    logits = lax.dot_general(
        w_ref[...], hs_ref[...], (((1,), (1,)), ((), ())),
        preferred_element_type=jnp.float32)
    scores = jax.nn.sigmoid(logits)
    scores_ref[:, pl.ds(i * TOK_BLK, TOK_BLK)] = lax.bitcast_convert_type(
        scores, jnp.int32)

    @pl.when(i == N_BLK - 1)
    def _():
        bits = scores_ref[...]

        def step(_, lohi):
            lo, hi = lohi
            mid = (lo + hi) >> 1
            cnt = jnp.sum((bits >= mid).astype(jnp.int32), axis=1,
                          keepdims=True)
            ge = cnt >= CAP
            return jnp.where(ge, mid, lo), jnp.where(ge, hi, mid)

        lo0 = jnp.zeros((N_EXPERTS, 1), jnp.int32)
        hi0 = jnp.full((N_EXPERTS, 1), 0x3F800001, jnp.int32)
        lo, _ = lax.fori_loop(0, 30, step, (lo0, hi0))
        cnt = jnp.sum((bits >= lo).astype(jnp.int32), axis=1, keepdims=True)
        nch = (cnt + 15) >> 4
        lane = lax.broadcasted_iota(jnp.int32, (N_EXPERTS, 128), 1)
        tbits_ref[...] = jnp.where(lane < 16, lo, nch)


_tc_gate = pl.pallas_call(
    _tc_gate_body,
    grid=(N_BLK,),
    in_specs=[
        pl.BlockSpec((TOK_BLK, EMBED), lambda i: (i, 0)),
        pl.BlockSpec((N_EXPERTS, EMBED), lambda i: (0, 0)),
    ],
    out_specs=[
        pl.BlockSpec((N_EXPERTS, N_TOKENS), lambda i: (0, 0)),
        pl.BlockSpec((N_EXPERTS, 128), lambda i: (0, 0)),
    ],
    out_shape=[
        jax.ShapeDtypeStruct((N_EXPERTS, N_TOKENS), jnp.int32),
        jax.ShapeDtypeStruct((N_EXPERTS, 128), jnp.int32),
    ],
)


def _radix_pass(shift, src_k, src_v, dst_k, dst_v, hist, offs, nch):
    zeros16 = jnp.zeros((16,), jnp.int32)
    hist[pl.ds(0, 16)] = zeros16
    hist[pl.ds(16, 16)] = zeros16

    def count_body(i, _):
        k = src_k[pl.ds(i * 16, 16)]
        d = 31 - ((k >> shift) & 31)
        occ, lastm = plsc.scan_count(d)
        plsc.addupdate_scatter(hist, [d], occ, mask=lastm)
        return 0

    lax.fori_loop(0, nch, count_body, 0, unroll=False)

    h0 = hist[pl.ds(0, 16)]
    h1 = hist[pl.ds(16, 16)]
    c0 = plsc.cumsum(h0)
    c1 = plsc.cumsum(h1)
    hist[pl.ds(0, 16)] = c0
    tot0 = plsc.load_gather(hist, [jnp.full((16,), 15, jnp.int32)])
    offs[pl.ds(0, 16)] = c0 - h0
    offs[pl.ds(16, 16)] = c1 - h1 + tot0

    def perm_body(i, _):
        k = src_k[pl.ds(i * 16, 16)]
        v = src_v[pl.ds(i * 16, 16)]
        d = 31 - ((k >> shift) & 31)
        occ, lastm = plsc.scan_count(d)
        base = plsc.load_gather(offs, [d])
        dest = base + occ - 1
        plsc.store_scatter(dst_k, [dest], k)
        plsc.store_scatter(dst_v, [dest], v)
        plsc.addupdate_scatter(offs, [d], occ, mask=lastm)
        return 0

    lax.fori_loop(0, nch, perm_body, 0, unroll=False)


def _sc_topk_body(scores_hbm, tbits_hbm, ids_hbm, obits_hbm,
                  s_v, t_v, ka, va, kb, vb, hist, offs):
    info = plsc.get_sparse_core_info()
    wid = lax.axis_index("s") * info.num_cores + lax.axis_index("c")

    @pl.when(wid < N_EXPERTS)
    def _():
        e = wid
        pltpu.sync_copy(scores_hbm.at[e], s_v)
        pltpu.sync_copy(tbits_hbm.at[e], t_v)
        t = t_v[pl.ds(0, 16)]
        lanes = lax.iota(jnp.int32, 16)

        def comp_body(i, off):
            b = s_v[pl.ds(i * 16, 16)]
            m = b >= t
            pos = off + plsc.cumsum(m.astype(jnp.int32)) - 1
            plsc.store_scatter(ka, [pos], b, mask=m)
            plsc.store_scatter(va, [pos], lanes + i * 16, mask=m)
            return off + plsc.all_reduce_population_count(m)

        off = lax.fori_loop(0, N_TOKENS // 16, comp_body,
                            jnp.zeros((16,), jnp.int32), unroll=False)
        zeros16 = jnp.zeros((16,), jnp.int32)
        plsc.store_scatter(ka, [off + lanes], zeros16)
        plsc.store_scatter(va, [off + lanes], zeros16)
        nchv = t_v[pl.ds(16, 16)]
        nch = jnp.minimum(nchv[0], SELCAP // 16)

        _radix_pass(0, ka, va, kb, vb, hist, offs, nch)
        _radix_pass(5, kb, vb, ka, va, hist, offs, nch)
        _radix_pass(10, ka, va, kb, vb, hist, offs, nch)
        _radix_pass(15, kb, vb, ka, va, hist, offs, nch)
        _radix_pass(20, ka, va, kb, vb, hist, offs, nch)
        _radix_pass(25, kb, vb, ka, va, hist, offs, nch)

        pltpu.sync_copy(va.at[pl.ds(0, CAP)], ids_hbm.at[e])
        pltpu.sync_copy(ka.at[pl.ds(0, CAP)], obits_hbm.at[e])


_sc_topk = pl.kernel(
    _sc_topk_body,
    out_type=(
        jax.ShapeDtypeStruct((N_EXPERTS, CAP), jnp.int32),
        jax.ShapeDtypeStruct((N_EXPERTS, CAP), jnp.int32),
    ),
    mesh=plsc.VectorSubcoreMesh(core_axis_name="c", subcore_axis_name="s"),
    compiler_params=pltpu.CompilerParams(needs_layout_passes=False),
    scratch_types=[
        pltpu.VMEM((N_TOKENS,), jnp.int32),
        pltpu.VMEM((128,), jnp.int32),
        pltpu.VMEM((SELCAP,), jnp.int32),
        pltpu.VMEM((SELCAP,), jnp.int32),
        pltpu.VMEM((SELCAP,), jnp.int32),
        pltpu.VMEM((SELCAP,), jnp.int32),
        pltpu.VMEM((32,), jnp.int32),
        pltpu.VMEM((32,), jnp.int32),
    ],
)


def kernel(hidden_states, weight):
    hs = hidden_states.reshape(-1, EMBED)
    score_bits, tbits = _tc_gate(hs, weight)
    topk_ids, topk_bits = _sc_topk(score_bits, tbits)
    topk_score = lax.bitcast_convert_type(topk_bits, jnp.float32)
    return topk_ids, topk_score


# speedup vs baseline: 1.5423x; 1.1687x over previous
import functools
import math

import jax
import jax.numpy as jnp
from jax import lax
from jax.experimental import pallas as pl
from jax.experimental.pallas import tpu as pltpu
from jax.experimental.pallas import tpu_sc as plsc

N_EXPERTS = 16
N_TOKENS = 8192
EMBED = 2048
CAP = 1024
TOK_BLK = 512
N_BLK = N_TOKENS // TOK_BLK
SELCAP = N_TOKENS + 16


def _tc_gate_body(hs_ref, w_ref, scores_ref, tbits_ref):
    i = pl.program_id(0)
    logits = lax.dot_general(
        w_ref[...], hs_ref[...], (((1,), (1,)), ((), ())),
        preferred_element_type=jnp.float32)
    scores = jax.nn.sigmoid(logits)
    scores_ref[:, pl.ds(i * TOK_BLK, TOK_BLK)] = lax.bitcast_convert_type(
        scores, jnp.int32)

    @pl.when(i == N_BLK - 1)
    def _():
        bits = scores_ref[...]

        def step(_, lohi):
            lo, hi = lohi
            mid = (lo + hi) >> 1
            cnt = jnp.sum((bits >= mid).astype(jnp.int32), axis=1,
                          keepdims=True)
            ge = cnt >= CAP
            return jnp.where(ge, mid, lo), jnp.where(ge, hi, mid)

        lo0 = jnp.zeros((N_EXPERTS, 1), jnp.int32)
        hi0 = jnp.full((N_EXPERTS, 1), 0x3F800001, jnp.int32)
        lo, _ = lax.fori_loop(0, 30, step, (lo0, hi0))
        cnt = jnp.sum((bits >= lo).astype(jnp.int32), axis=1, keepdims=True)
        nch = (cnt + 15) >> 4
        lane = lax.broadcasted_iota(jnp.int32, (N_EXPERTS, 128), 1)
        tbits_ref[...] = jnp.where(lane < 16, lo, nch)


_tc_gate = pl.pallas_call(
    _tc_gate_body,
    grid=(N_BLK,),
    in_specs=[
        pl.BlockSpec((TOK_BLK, EMBED), lambda i: (i, 0)),
        pl.BlockSpec((N_EXPERTS, EMBED), lambda i: (0, 0)),
    ],
    out_specs=[
        pl.BlockSpec((N_EXPERTS, N_TOKENS), lambda i: (0, 0)),
        pl.BlockSpec((N_EXPERTS, 128), lambda i: (0, 0)),
    ],
    out_shape=[
        jax.ShapeDtypeStruct((N_EXPERTS, N_TOKENS), jnp.int32),
        jax.ShapeDtypeStruct((N_EXPERTS, 128), jnp.int32),
    ],
)


def _radix_pass(shift_cur, shift_next, src_k, src_v, dst_k, dst_v,
                hcur, hnext, offs, nch):
    zeros16 = jnp.zeros((16,), jnp.int32)
    h0 = hcur[pl.ds(0, 16)]
    h1 = hcur[pl.ds(16, 16)]
    c0 = plsc.cumsum(h0)
    c1 = plsc.cumsum(h1)
    hcur[pl.ds(0, 16)] = c0
    tot0 = plsc.load_gather(hcur, [jnp.full((16,), 15, jnp.int32)])
    offs[pl.ds(0, 16)] = c0 - h0
    offs[pl.ds(16, 16)] = c1 - h1 + tot0
    if shift_next >= 0:
        hnext[pl.ds(0, 16)] = zeros16
        hnext[pl.ds(16, 16)] = zeros16

    def perm_body(i, _):
        k = src_k[pl.ds(i * 16, 16)]
        v = src_v[pl.ds(i * 16, 16)]
        d = 31 - ((k >> shift_cur) & 31)
        occ, lastm = plsc.scan_count(d)
        base = plsc.load_gather(offs, [d])
        dest = base + occ - 1
        plsc.store_scatter(dst_k, [dest], k)
        plsc.store_scatter(dst_v, [dest], v)
        plsc.addupdate_scatter(offs, [d], occ, mask=lastm)
        if shift_next >= 0:
            d2 = 31 - ((k >> shift_next) & 31)
            occ2, lastm2 = plsc.scan_count(d2)
            plsc.addupdate_scatter(hnext, [d2], occ2, mask=lastm2)
        return 0

    lax.fori_loop(0, nch, perm_body, 0, unroll=False)


def _sc_topk_body(scores_hbm, tbits_hbm, ids_hbm, obits_hbm,
                  s_v, t_v, ka, va, kb, vb, hist, hist2, offs):
    info = plsc.get_sparse_core_info()
    wid = lax.axis_index("s") * info.num_cores + lax.axis_index("c")

    @pl.when(wid < N_EXPERTS)
    def _():
        e = wid
        pltpu.sync_copy(scores_hbm.at[e], s_v)
        pltpu.sync_copy(tbits_hbm.at[e], t_v)
        t = t_v[pl.ds(0, 16)]
        lanes = lax.iota(jnp.int32, 16)
        zeros16 = jnp.zeros((16,), jnp.int32)
        hist[pl.ds(0, 16)] = zeros16
        hist[pl.ds(16, 16)] = zeros16

        @plsc.parallel_loop(0, N_TOKENS // 16, carry=jnp.zeros((16,),
                                                              jnp.int32))
        def comp_body(i, off):
            b = s_v[pl.ds(i * 16, 16)]
            m = b >= t
            pos = off + plsc.cumsum(m.astype(jnp.int32)) - 1
            plsc.store_scatter(ka, [pos], b, mask=m)
            plsc.store_scatter(va, [pos], lanes + i * 16, mask=m)
            d0 = 31 - (b & 31)
            occ0, lastm0 = plsc.scan_count(d0, mask=m)
            plsc.addupdate_scatter(hist, [d0], occ0, mask=lastm0)
            return off + plsc.all_reduce_population_count(m)

        off = comp_body
        plsc.store_scatter(ka, [off + lanes], zeros16)
        plsc.store_scatter(va, [off + lanes], zeros16)
        nchv = t_v[pl.ds(16, 16)]
        nch = jnp.minimum(nchv[0], SELCAP // 16)
        padv = nchv * 16 - off
        plsc.addupdate_scatter(hist, [jnp.full((16,), 31, jnp.int32)],
                               padv, mask=lanes == 0)

        _radix_pass(0, 5, ka, va, kb, vb, hist, hist2, offs, nch)
        _radix_pass(5, 10, kb, vb, ka, va, hist2, hist, offs, nch)
        _radix_pass(10, 15, ka, va, kb, vb, hist, hist2, offs, nch)
        _radix_pass(15, 20, kb, vb, ka, va, hist2, hist, offs, nch)
        _radix_pass(20, 25, ka, va, kb, vb, hist, hist2, offs, nch)
        _radix_pass(25, -1, kb, vb, ka, va, hist2, hist, offs, nch)

        pltpu.sync_copy(va.at[pl.ds(0, CAP)], ids_hbm.at[e])
        pltpu.sync_copy(ka.at[pl.ds(0, CAP)], obits_hbm.at[e])


_sc_topk = pl.kernel(
    _sc_topk_body,
    out_type=(
        jax.ShapeDtypeStruct((N_EXPERTS, CAP), jnp.int32),
        jax.ShapeDtypeStruct((N_EXPERTS, CAP), jnp.int32),
    ),
    mesh=plsc.VectorSubcoreMesh(core_axis_name="c", subcore_axis_name="s"),
    compiler_params=pltpu.CompilerParams(needs_layout_passes=False),
    scratch_types=[
        pltpu.VMEM((N_TOKENS,), jnp.int32),
        pltpu.VMEM((128,), jnp.int32),
        pltpu.VMEM((SELCAP,), jnp.int32),
        pltpu.VMEM((SELCAP,), jnp.int32),
        pltpu.VMEM((SELCAP,), jnp.int32),
        pltpu.VMEM((SELCAP,), jnp.int32),
        pltpu.VMEM((32,), jnp.int32),
        pltpu.VMEM((32,), jnp.int32),
        pltpu.VMEM((32,), jnp.int32),
    ],
)


def kernel(hidden_states, weight):
    hs = hidden_states.reshape(-1, EMBED)
    score_bits, tbits = _tc_gate(hs, weight)
    topk_ids, topk_bits = _sc_topk(score_bits, tbits)
    topk_score = lax.bitcast_convert_type(topk_bits, jnp.float32)
    return topk_ids, topk_score


# speedup vs baseline: 1.5667x; 1.0159x over previous
import functools
import math

import jax
import jax.numpy as jnp
from jax import lax
from jax.experimental import pallas as pl
from jax.experimental.pallas import tpu as pltpu
from jax.experimental.pallas import tpu_sc as plsc

N_EXPERTS = 16
N_TOKENS = 8192
EMBED = 2048
CAP = 1024
TOK_BLK = 512
N_BLK = N_TOKENS // TOK_BLK
SELCAP = N_TOKENS + 16


def _tc_gate_body(hs_ref, w_ref, scores_ref, tbits_ref):
    i = pl.program_id(0)
    logits = lax.dot_general(
        w_ref[...], hs_ref[...], (((1,), (1,)), ((), ())),
        preferred_element_type=jnp.float32)
    scores = jax.nn.sigmoid(logits)
    scores_ref[:, pl.ds(i * TOK_BLK, TOK_BLK)] = lax.bitcast_convert_type(
        scores, jnp.int32)

    @pl.when(i == N_BLK - 1)
    def _():
        bits = scores_ref[...]

        def step(_, lohi):
            lo, hi = lohi
            mid = (lo + hi) >> 1
            cnt = jnp.sum((bits >= mid).astype(jnp.int32), axis=1,
                          keepdims=True)
            ge = cnt >= CAP
            return jnp.where(ge, mid, lo), jnp.where(ge, hi, mid)

        lo0 = jnp.zeros((N_EXPERTS, 1), jnp.int32)
        hi0 = jnp.full((N_EXPERTS, 1), 0x3F800001, jnp.int32)
        lo, _ = lax.fori_loop(0, 30, step, (lo0, hi0))
        cnt = jnp.sum((bits >= lo).astype(jnp.int32), axis=1, keepdims=True)
        nch = (cnt + 15) >> 4
        lane = lax.broadcasted_iota(jnp.int32, (N_EXPERTS, 128), 1)
        tbits_ref[...] = jnp.where(lane < 16, lo, nch)


_tc_gate = pl.pallas_call(
    _tc_gate_body,
    grid=(N_BLK,),
    in_specs=[
        pl.BlockSpec((TOK_BLK, EMBED), lambda i: (i, 0)),
        pl.BlockSpec((N_EXPERTS, EMBED), lambda i: (0, 0)),
    ],
    out_specs=[
        pl.BlockSpec((N_EXPERTS, N_TOKENS), lambda i: (0, 0)),
        pl.BlockSpec((N_EXPERTS, 128), lambda i: (0, 0)),
    ],
    out_shape=[
        jax.ShapeDtypeStruct((N_EXPERTS, N_TOKENS), jnp.int32),
        jax.ShapeDtypeStruct((N_EXPERTS, 128), jnp.int32),
    ],
)


def _radix_pass(shift_cur, shift_next, src_k, src_v, dst_k, dst_v,
                hcur, hnext, offs, nch):
    zeros16 = jnp.zeros((16,), jnp.int32)
    h0 = hcur[pl.ds(0, 16)]
    h1 = hcur[pl.ds(16, 16)]
    c0 = plsc.cumsum(h0)
    c1 = plsc.cumsum(h1)
    hcur[pl.ds(0, 16)] = c0
    tot0 = plsc.load_gather(hcur, [jnp.full((16,), 15, jnp.int32)])
    offs[pl.ds(0, 16)] = c0 - h0
    offs[pl.ds(16, 16)] = c1 - h1 + tot0
    if shift_next >= 0:
        hnext[pl.ds(0, 16)] = zeros16
        hnext[pl.ds(16, 16)] = zeros16

    def perm_body(i, _):
        k = src_k[pl.ds(i * 16, 16)]
        v = src_v[pl.ds(i * 16, 16)]
        d = 31 - ((k >> shift_cur) & 31)
        occ, lastm = plsc.scan_count(d)
        base = plsc.load_gather(offs, [d])
        dest = base + occ - 1
        plsc.store_scatter(dst_k, [dest], k)
        plsc.store_scatter(dst_v, [dest], v)
        plsc.addupdate_scatter(offs, [d], occ, mask=lastm)
        if shift_next >= 0:
            d2 = 31 - ((k >> shift_next) & 31)
            occ2, lastm2 = plsc.scan_count(d2)
            plsc.addupdate_scatter(hnext, [d2], occ2, mask=lastm2)
        return 0

    lax.fori_loop(0, nch, perm_body, 0)


def _sc_topk_body(scores_hbm, tbits_hbm, ids_hbm, obits_hbm,
                  s_v, t_v, ka, va, kb, vb, hist, hist2, offs):
    info = plsc.get_sparse_core_info()
    wid = lax.axis_index("s") * info.num_cores + lax.axis_index("c")

    @pl.when(wid < N_EXPERTS)
    def _():
        e = wid
        pltpu.sync_copy(scores_hbm.at[e], s_v)
        pltpu.sync_copy(tbits_hbm.at[e], t_v)
        t = t_v[pl.ds(0, 16)]
        lanes = lax.iota(jnp.int32, 16)
        zeros16 = jnp.zeros((16,), jnp.int32)
        hist[pl.ds(0, 16)] = zeros16
        hist[pl.ds(16, 16)] = zeros16

        @plsc.parallel_loop(0, N_TOKENS // 16, unroll=4,
                            carry=jnp.zeros((16,), jnp.int32))
        def comp_body(i, off):
            b = s_v[pl.ds(i * 16, 16)]
            m = b >= t
            pos = off + plsc.cumsum(m.astype(jnp.int32)) - 1
            plsc.store_scatter(ka, [pos], b, mask=m)
            plsc.store_scatter(va, [pos], lanes + i * 16, mask=m)
            d0 = 31 - (b & 31)
            occ0, lastm0 = plsc.scan_count(d0, mask=m)
            plsc.addupdate_scatter(hist, [d0], occ0, mask=lastm0)
            return off + plsc.all_reduce_population_count(m)

        off = comp_body
        plsc.store_scatter(ka, [off + lanes], zeros16)
        plsc.store_scatter(va, [off + lanes], zeros16)
        nchv = t_v[pl.ds(16, 16)]
        nch = jnp.minimum(nchv[0], SELCAP // 16)
        padv = nchv * 16 - off
        plsc.addupdate_scatter(hist, [jnp.full((16,), 31, jnp.int32)],
                               padv, mask=lanes == 0)

        _radix_pass(0, 5, ka, va, kb, vb, hist, hist2, offs, nch)
        _radix_pass(5, 10, kb, vb, ka, va, hist2, hist, offs, nch)
        _radix_pass(10, 15, ka, va, kb, vb, hist, hist2, offs, nch)
        _radix_pass(15, 20, kb, vb, ka, va, hist2, hist, offs, nch)
        _radix_pass(20, 25, ka, va, kb, vb, hist, hist2, offs, nch)
        _radix_pass(25, -1, kb, vb, ka, va, hist2, hist, offs, nch)

        pltpu.sync_copy(va.at[pl.ds(0, CAP)], ids_hbm.at[e])
        pltpu.sync_copy(ka.at[pl.ds(0, CAP)], obits_hbm.at[e])


_sc_topk = pl.kernel(
    _sc_topk_body,
    out_type=(
        jax.ShapeDtypeStruct((N_EXPERTS, CAP), jnp.int32),
        jax.ShapeDtypeStruct((N_EXPERTS, CAP), jnp.int32),
    ),
    mesh=plsc.VectorSubcoreMesh(core_axis_name="c", subcore_axis_name="s"),
    compiler_params=pltpu.CompilerParams(needs_layout_passes=False),
    scratch_types=[
        pltpu.VMEM((N_TOKENS,), jnp.int32),
        pltpu.VMEM((128,), jnp.int32),
        pltpu.VMEM((SELCAP,), jnp.int32),
        pltpu.VMEM((SELCAP,), jnp.int32),
        pltpu.VMEM((SELCAP,), jnp.int32),
        pltpu.VMEM((SELCAP,), jnp.int32),
        pltpu.VMEM((32,), jnp.int32),
        pltpu.VMEM((32,), jnp.int32),
        pltpu.VMEM((32,), jnp.int32),
    ],
)


def kernel(hidden_states, weight):
    hs = hidden_states.reshape(-1, EMBED)
    score_bits, tbits = _tc_gate(hs, weight)
    topk_ids, topk_bits = _sc_topk(score_bits, tbits)
    topk_score = lax.bitcast_convert_type(topk_bits, jnp.float32)
    return topk_ids, topk_score
